# baseline (speedup 1.0000x reference)
import functools

import jax
import jax.numpy as jnp
from jax import lax
from jax.experimental import pallas as pl
from jax.experimental.pallas import tpu as pltpu
from jax.experimental.pallas import tpu_sc as plsc

E = 8
D = 768
H = 768
KM1 = 7
K = 8
B = 32768
T = 512
P = B + E * T
NTILES = P // T
OUTW = 16

NC = 2
NS = 16
NW = NC * NS


@functools.cache
def _make_sc_row_gather(n_src, n_out, d, chunk, nbuf):
    rows_per_w = n_out // NW
    n_chunks = rows_per_w // chunk
    assert rows_per_w % chunk == 0 and n_out % NW == 0 and chunk <= 128
    mesh = plsc.VectorSubcoreMesh(core_axis_name="c", subcore_axis_name="s")

    @functools.partial(
        pl.kernel,
        out_type=jax.ShapeDtypeStruct((n_out, d), jnp.float32),
        mesh=mesh,
        scratch_types=(
            [pltpu.VMEM((rows_per_w,), jnp.int32)]
            + [pltpu.VMEM((chunk, d), jnp.float32) for _ in range(nbuf)]
            + [pltpu.SemaphoreType.DMA for _ in range(2 * nbuf)]
        ),
    )
    def gather_k(table_hbm, idx_hbm, out_hbm, idx_v, *rest):
        bufs = rest[:nbuf]
        gsems = rest[nbuf:2 * nbuf]
        wsems = rest[2 * nbuf:]
        wid = lax.axis_index("s") * NC + lax.axis_index("c")
        base = wid * rows_per_w
        pltpu.sync_copy(idx_hbm.at[pl.ds(base, rows_per_w)], idx_v)

        def start_gather(c, b):
            return pltpu.async_copy(
                table_hbm.at[idx_v.at[pl.ds(c * chunk, chunk)]],
                bufs[b], gsems[b])

        def start_write(c, b):
            return pltpu.async_copy(
                bufs[b], out_hbm.at[pl.ds(base + c * chunk, chunk)], wsems[b])

        cps_g = [start_gather(b, b) for b in range(min(nbuf, n_chunks))]
        cps_w = [None] * nbuf
        for c in range(n_chunks):
            b = c % nbuf
            cps_g[b].wait()
            cps_w[b] = start_write(c, b)
            nxt = c + nbuf
            if nxt < n_chunks:
                cps_w[b].wait()
                cps_g[b] = start_gather(nxt, b)
        for c in range(max(0, n_chunks - nbuf), n_chunks):
            cps_w[c % nbuf].wait()

    return gather_k


@functools.cache
def _make_sc_reorder(n_src, n_out, d, chunk, nbuf):
    n_rows = n_out
    rows_per_w = n_rows // NW
    n_chunks = rows_per_w // chunk
    assert rows_per_w % chunk == 0 and chunk <= 128
    mesh = plsc.VectorSubcoreMesh(core_axis_name="c", subcore_axis_name="s")

    @functools.partial(
        pl.kernel,
        out_type=jax.ShapeDtypeStruct((n_out, d), jnp.float32),
        mesh=mesh,
        scratch_types=(
            [pltpu.VMEM((n_chunks, chunk), jnp.int32),
             pltpu.VMEM((n_chunks, chunk), jnp.int32)]
            + [pltpu.VMEM((chunk, d), jnp.float32) for _ in range(nbuf)]
            + [pltpu.SemaphoreType.DMA for _ in range(2 * nbuf)]
        ),
        compiler_params=pltpu.CompilerParams(use_tc_tiling_on_sc=False),
    )
    def reorder_k(table_hbm, src_hbm, dst_hbm, out_hbm, src_v, dst_v, *rest):
        bufs = rest[:nbuf]
        gsems = rest[nbuf:2 * nbuf]
        wsems = rest[2 * nbuf:]
        wid = lax.axis_index("s") * NC + lax.axis_index("c")
        pltpu.sync_copy(src_hbm.at[pl.ds(wid * n_chunks, n_chunks)], src_v)
        pltpu.sync_copy(dst_hbm.at[pl.ds(wid * n_chunks, n_chunks)], dst_v)

        def start_gather(c, b):
            return pltpu.async_copy(table_hbm.at[src_v.at[c]], bufs[b],
                                    gsems[b])

        def start_write(c, b):
            return pltpu.async_copy(bufs[b], out_hbm.at[dst_v.at[c]],
                                    wsems[b])

        cps_g = [start_gather(b, b) for b in range(min(nbuf, n_chunks))]
        cps_w = [None] * nbuf
        for c in range(n_chunks):
            b = c % nbuf
            cps_g[b].wait()
            cps_w[b] = start_write(c, b)
            nxt = c + nbuf
            if nxt < n_chunks:
                cps_w[b].wait()
                cps_g[b] = start_gather(nxt, b)
        for c in range(max(0, n_chunks - nbuf), n_chunks):
            cps_w[c % nbuf].wait()

    return reorder_k


def _tc_tile_kernel(eid_ref, xs_ref, w1_ref, b1_ref, w2_ref, b2_ref, out_ref):
    xb = xs_ref[...]
    h = jnp.dot(xb, w1_ref[0], preferred_element_type=jnp.float32)
    h = jnp.maximum(h + b1_ref[0], 0.0)
    logits8 = jnp.dot(h, w2_ref[0], preferred_element_type=jnp.float32)
    logits8 = logits8 + b2_ref[0]
    q = jax.nn.sigmoid(logits8[:, :KM1])
    one = jnp.ones((T, 1), jnp.float32)
    zero = jnp.zeros((T, 1), jnp.float32)
    qs = jnp.concatenate([one, q], axis=1)
    qe = jnp.concatenate([q, zero], axis=1)
    probs = jnp.maximum(qs - qe, 1e-8)
    probs = probs / jnp.maximum(jnp.sum(probs, axis=1, keepdims=True), 1e-8)
    out_ref[:, 0:8] = logits8
    out_ref[:, 8:16] = probs


def _tc_experts(xs, w1, b1r, w2p, b2r, tile_eid):
    grid_spec = pltpu.PrefetchScalarGridSpec(
        num_scalar_prefetch=1,
        grid=(NTILES,),
        in_specs=[
            pl.BlockSpec((T, D), lambda i, eid: (i, 0)),
            pl.BlockSpec((1, D, H), lambda i, eid: (eid[i], 0, 0)),
            pl.BlockSpec((1, 1, H), lambda i, eid: (eid[i], 0, 0)),
            pl.BlockSpec((1, H, K), lambda i, eid: (eid[i], 0, 0)),
            pl.BlockSpec((1, 1, K), lambda i, eid: (eid[i], 0, 0)),
        ],
        out_specs=pl.BlockSpec((T, OUTW), lambda i, eid: (i, 0)),
    )
    return pl.pallas_call(
        _tc_tile_kernel,
        grid_spec=grid_spec,
        out_shape=jax.ShapeDtypeStruct((P, OUTW), jnp.float32),
        compiler_params=pltpu.CompilerParams(
            dimension_semantics=("arbitrary",),
        ),
    )(tile_eid, xs, w1, b1r, w2p, b2r)


def kernel(x, size_idx, W1, b1, W2, b2):
    si = size_idx.astype(jnp.int32)

    iota_b = jnp.arange(B, dtype=jnp.int32)
    sorted_e, perm = lax.sort((si, iota_b), num_keys=1)
    counts = jnp.sum(sorted_e[:, None] == jnp.arange(E, dtype=jnp.int32)[None, :],
                     axis=0, dtype=jnp.int32)
    padded = ((counts + T - 1) // T) * T
    seg_end = jnp.cumsum(padded).astype(jnp.int32)
    pstart = seg_end - padded
    off = jnp.cumsum(counts).astype(jnp.int32) - counts
    ppos = pstart[sorted_e] + (iota_b - off[sorted_e])

    tile_start = jnp.arange(NTILES, dtype=jnp.int32) * T
    tile_eid = jnp.sum(tile_start[:, None] >= seg_end[None, :],
                       axis=1, dtype=jnp.int32)
    tile_eid = jnp.minimum(tile_eid, E - 1)

    e_full = jnp.broadcast_to(tile_eid[:, None], (NTILES, T)).reshape(P)
    jj = jnp.arange(P, dtype=jnp.int32)
    r = jj - pstart[e_full]
    sidx = jnp.where(r < counts[e_full], off[e_full] + r, B)
    perm_pad = jnp.concatenate([perm, jnp.zeros((1,), jnp.int32)])
    row_ids = perm_pad[sidx]

    xs = _make_sc_row_gather(B, P, D, 32, 4)(x, row_ids)

    b1r = b1.reshape(E, 1, H)
    w2p = jnp.pad(W2, ((0, 0), (0, 0), (0, 1)))
    b2r = jnp.pad(b2, ((0, 0), (0, 1))).reshape(E, 1, K)
    packed = _tc_experts(xs, W1, b1r, w2p, b2r, tile_eid)

    rc = B // (NW * 128)
    src2 = ppos.reshape(NW * rc, 128)
    dst2 = perm.reshape(NW * rc, 128)
    out16 = _make_sc_reorder(P, B, OUTW, 128, 2)(packed, src2, dst2)
    logits = out16[:, :KM1]
    probs = out16[:, 8:16]
    return (logits, probs)


# speedup vs baseline: 1.6277x; 1.6277x over previous
import functools

import jax
import jax.numpy as jnp
from jax import lax
from jax.experimental import pallas as pl
from jax.experimental.pallas import tpu as pltpu
from jax.experimental.pallas import tpu_sc as plsc

E = 8
D = 768
H = 768
KM1 = 7
K = 8
B = 32768
T = 256
P = B + E * T
NTILES = P // T
OUTW = 16

NC = 2
NS = 16
NW = NC * NS


@functools.cache
def _make_sc_row_gather(n_src, n_out, d, chunk, nbuf):
    rows_per_w = n_out // NW
    n_chunks = rows_per_w // chunk
    assert rows_per_w % chunk == 0 and n_out % NW == 0 and chunk <= 128
    mesh = plsc.VectorSubcoreMesh(core_axis_name="c", subcore_axis_name="s")

    @functools.partial(
        pl.kernel,
        out_type=jax.ShapeDtypeStruct((n_out, d), jnp.float32),
        mesh=mesh,
        scratch_types=(
            [pltpu.VMEM((rows_per_w,), jnp.int32)]
            + [pltpu.VMEM((chunk, d), jnp.float32) for _ in range(nbuf)]
            + [pltpu.SemaphoreType.DMA for _ in range(2 * nbuf)]
        ),
    )
    def gather_k(table_hbm, idx_hbm, out_hbm, idx_v, *rest):
        bufs = rest[:nbuf]
        gsems = rest[nbuf:2 * nbuf]
        wsems = rest[2 * nbuf:]
        wid = lax.axis_index("s") * NC + lax.axis_index("c")
        base = wid * rows_per_w
        pltpu.sync_copy(idx_hbm.at[pl.ds(base, rows_per_w)], idx_v)

        def start_gather(c, b):
            return pltpu.async_copy(
                table_hbm.at[idx_v.at[pl.ds(c * chunk, chunk)]],
                bufs[b], gsems[b])

        def start_write(c, b):
            return pltpu.async_copy(
                bufs[b], out_hbm.at[pl.ds(base + c * chunk, chunk)], wsems[b])

        cps_g = [start_gather(b, b) for b in range(min(nbuf, n_chunks))]
        cps_w = [None] * nbuf
        for c in range(n_chunks):
            b = c % nbuf
            cps_g[b].wait()
            cps_w[b] = start_write(c, b)
            nxt = c + nbuf
            if nxt < n_chunks:
                cps_w[b].wait()
                cps_g[b] = start_gather(nxt, b)
        for c in range(max(0, n_chunks - nbuf), n_chunks):
            cps_w[c % nbuf].wait()

    return gather_k


@functools.cache
def _make_sc_reorder(n_src, n_out, d, chunk, nbuf):
    n_rows = n_out
    rows_per_w = n_rows // NW
    n_chunks = rows_per_w // chunk
    assert rows_per_w % chunk == 0 and chunk <= 128
    mesh = plsc.VectorSubcoreMesh(core_axis_name="c", subcore_axis_name="s")

    @functools.partial(
        pl.kernel,
        out_type=jax.ShapeDtypeStruct((n_out, d), jnp.float32),
        mesh=mesh,
        scratch_types=(
            [pltpu.VMEM((n_chunks, chunk), jnp.int32),
             pltpu.VMEM((n_chunks, chunk), jnp.int32)]
            + [pltpu.VMEM((chunk, d), jnp.float32) for _ in range(nbuf)]
            + [pltpu.SemaphoreType.DMA for _ in range(2 * nbuf)]
        ),
        compiler_params=pltpu.CompilerParams(use_tc_tiling_on_sc=False),
    )
    def reorder_k(table_hbm, src_hbm, dst_hbm, out_hbm, src_v, dst_v, *rest):
        bufs = rest[:nbuf]
        gsems = rest[nbuf:2 * nbuf]
        wsems = rest[2 * nbuf:]
        wid = lax.axis_index("s") * NC + lax.axis_index("c")
        pltpu.sync_copy(src_hbm.at[pl.ds(wid * n_chunks, n_chunks)], src_v)
        pltpu.sync_copy(dst_hbm.at[pl.ds(wid * n_chunks, n_chunks)], dst_v)

        def start_gather(c, b):
            return pltpu.async_copy(table_hbm.at[src_v.at[c]], bufs[b],
                                    gsems[b])

        def start_write(c, b):
            return pltpu.async_copy(bufs[b], out_hbm.at[dst_v.at[c]],
                                    wsems[b])

        cps_g = [start_gather(b, b) for b in range(min(nbuf, n_chunks))]
        cps_w = [None] * nbuf
        for c in range(n_chunks):
            b = c % nbuf
            cps_g[b].wait()
            cps_w[b] = start_write(c, b)
            nxt = c + nbuf
            if nxt < n_chunks:
                cps_w[b].wait()
                cps_g[b] = start_gather(nxt, b)
        for c in range(max(0, n_chunks - nbuf), n_chunks):
            cps_w[c % nbuf].wait()

    return reorder_k


def _tc_tile_kernel(eid_ref, xs_ref, w1_ref, b1_ref, w2_ref, b2_ref, out_ref):
    xb = xs_ref[...]
    h = jnp.dot(xb, w1_ref[0], preferred_element_type=jnp.float32)
    h = jnp.maximum(h + b1_ref[0], 0.0)
    logits8 = jnp.dot(h, w2_ref[0], preferred_element_type=jnp.float32)
    logits8 = logits8 + b2_ref[0]
    q = jax.nn.sigmoid(logits8[:, :KM1])
    one = jnp.ones((T, 1), jnp.float32)
    zero = jnp.zeros((T, 1), jnp.float32)
    qs = jnp.concatenate([one, q], axis=1)
    qe = jnp.concatenate([q, zero], axis=1)
    probs = jnp.maximum(qs - qe, 1e-8)
    probs = probs / jnp.maximum(jnp.sum(probs, axis=1, keepdims=True), 1e-8)
    out_ref[:, 0:8] = logits8
    out_ref[:, 8:16] = probs


def _tc_experts(xs, w1, b1r, w2p, b2r, tile_eid):
    grid_spec = pltpu.PrefetchScalarGridSpec(
        num_scalar_prefetch=1,
        grid=(NTILES,),
        in_specs=[
            pl.BlockSpec((T, D), lambda i, eid: (i, 0)),
            pl.BlockSpec((1, D, H), lambda i, eid: (eid[i], 0, 0)),
            pl.BlockSpec((1, 1, H), lambda i, eid: (eid[i], 0, 0)),
            pl.BlockSpec((1, H, K), lambda i, eid: (eid[i], 0, 0)),
            pl.BlockSpec((1, 1, K), lambda i, eid: (eid[i], 0, 0)),
        ],
        out_specs=pl.BlockSpec((T, OUTW), lambda i, eid: (i, 0)),
    )
    return pl.pallas_call(
        _tc_tile_kernel,
        grid_spec=grid_spec,
        out_shape=jax.ShapeDtypeStruct((P, OUTW), jnp.float32),
        compiler_params=pltpu.CompilerParams(
            dimension_semantics=("arbitrary",),
        ),
    )(tile_eid, xs, w1, b1r, w2p, b2r)


def kernel(x, size_idx, W1, b1, W2, b2):
    si = size_idx.astype(jnp.int32)

    iota_b = jnp.arange(B, dtype=jnp.int32)
    sorted_e, perm = lax.sort((si, iota_b), num_keys=1)
    counts = jnp.sum(sorted_e[:, None] == jnp.arange(E, dtype=jnp.int32)[None, :],
                     axis=0, dtype=jnp.int32)
    padded = ((counts + T - 1) // T) * T
    seg_end = jnp.cumsum(padded).astype(jnp.int32)
    pstart = seg_end - padded
    off = jnp.cumsum(counts).astype(jnp.int32) - counts
    ppos = pstart[sorted_e] + (iota_b - off[sorted_e])

    tile_start = jnp.arange(NTILES, dtype=jnp.int32) * T
    tile_eid = jnp.sum(tile_start[:, None] >= seg_end[None, :],
                       axis=1, dtype=jnp.int32)
    tile_eid = jnp.minimum(tile_eid, E - 1)

    e_full = jnp.broadcast_to(tile_eid[:, None], (NTILES, T)).reshape(P)
    jj = jnp.arange(P, dtype=jnp.int32)
    r = jj - pstart[e_full]
    valid = r < counts[e_full]
    sidx = jnp.where(valid, off[e_full] + r, B)
    perm_pad = jnp.concatenate([perm, jnp.zeros((1,), jnp.int32)])
    pad_src = jnp.where(jj < B, jj, jj - B)
    row_ids = jnp.where(valid, perm_pad[sidx], pad_src)

    xs = _make_sc_row_gather(B, P, D, 32, 4)(x, row_ids)

    b1r = b1.reshape(E, 1, H)
    w2p = jnp.pad(W2, ((0, 0), (0, 0), (0, 1)))
    b2r = jnp.pad(b2, ((0, 0), (0, 1))).reshape(E, 1, K)
    packed = _tc_experts(xs, W1, b1r, w2p, b2r, tile_eid)

    rc = B // (NW * 128)
    src2 = ppos.reshape(NW * rc, 128)
    dst2 = perm.reshape(NW * rc, 128)
    out16 = _make_sc_reorder(P, B, OUTW, 128, 2)(packed, src2, dst2)
    logits = out16[:, :KM1]
    probs = out16[:, 8:16]
    return (logits, probs)
